# baseline (speedup 1.0000x reference)
import jax
import jax.numpy as jnp
from jax.experimental import pallas as pl
from jax.experimental.pallas import tpu as pltpu

_ROWS = 1024
_COLS = 27278
_BLOCK_ROWS = 64
_GRID = _ROWS // _BLOCK_ROWS


def _mse_block_kernel(out_ref, tgt_ref, sum_ref, cnt_ref):
    o = out_ref[...]
    t = tgt_ref[...]
    mask = t != -1.0
    d = o - t
    sq = jnp.where(mask, d * d, 0.0)
    sum_ref[...] = jnp.sum(sq).reshape(1, 1, 1)
    cnt_ref[...] = jnp.sum(mask.astype(jnp.float32)).reshape(1, 1, 1)


def kernel(output, target):
    in_spec = pl.BlockSpec((_BLOCK_ROWS, _COLS), lambda i: (i, 0))
    out_spec = pl.BlockSpec((1, 1, 1), lambda i: (i, 0, 0))
    partial_sums, partial_cnts = pl.pallas_call(
        _mse_block_kernel,
        grid=(_GRID,),
        in_specs=[in_spec, in_spec],
        out_specs=[out_spec, out_spec],
        out_shape=[
            jax.ShapeDtypeStruct((_GRID, 1, 1), jnp.float32),
            jax.ShapeDtypeStruct((_GRID, 1, 1), jnp.float32),
        ],
        compiler_params=pltpu.CompilerParams(
            dimension_semantics=("parallel",),
        ),
    )(output, target)
    return jnp.sum(partial_sums) / jnp.sum(partial_cnts)
